# baseline (speedup 1.0000x reference)
import functools

import jax
import jax.numpy as jnp
from jax import lax
from jax.experimental import pallas as pl
from jax.experimental.pallas import tpu as pltpu
from jax.experimental.pallas import tpu_sc as plsc

N = 10000
E = 320000
D = 128
H = 64
ATT = 16
G = 128
K = 3
ALPHA = 0.1

NC = 2
NS = 16
HC = H // NC
CH = 128
NBUF = 8
NGRP = 20
NCHT = NBUF * NGRP
PAD_E = NS * NCHT * CH
NR = 10240
RPT = NR // NS
NB_INIT = RPT // CH
DUMP0 = N + 48


@functools.cache
def _mesh():
    return plsc.VectorSubcoreMesh(
        core_axis_name="c", subcore_axis_name="s",
        num_cores=NC, num_subcores=NS)


def _splat(vec_ref, idx):
    return plsc.load_gather(vec_ref, [jnp.full((16,), idx, jnp.int32)])


def _appnp_body(y0c, srcp, dstp, y3c, gbuf, g0b, sidx, didx, rows, onesv,
                degv, d2v, f1v, f2v, acc, acc1, isem, gsem, ssem):
    c = lax.axis_index("c")
    s = lax.axis_index("s")

    pltpu.async_copy(srcp.at[pl.ds(s * NCHT, NCHT)], sidx, isem)
    pltpu.async_copy(dstp.at[pl.ds(s * NCHT, NCHT)], didx, isem)

    for i in range(CH // 16):
        onesv[pl.ds(i * 16, 16)] = jnp.ones((16,), jnp.float32)
    for i in range(RPT // 16):
        degv[pl.ds(i * 16, 16)] = jnp.zeros((16,), jnp.float32)
    pltpu.sync_copy(degv, acc1.at[pl.ds(s * RPT, RPT)])
    pltpu.make_async_copy(srcp.at[pl.ds(0, NCHT)], sidx, isem).wait()
    pltpu.make_async_copy(dstp.at[pl.ds(0, NCHT)], didx, isem).wait()
    plsc.subcore_barrier()

    def dfire(j, _):
        pltpu.async_copy(onesv, acc1.at[didx.at[j]], isem, add=True)
        return _

    def ddrain(j, _):
        pltpu.make_async_copy(onesv, acc1.at[didx.at[0]], isem).wait()
        return _

    lax.fori_loop(0, NCHT, dfire, None)
    lax.fori_loop(0, NCHT, ddrain, None)
    plsc.subcore_barrier()

    pltpu.sync_copy(acc1.at[pl.ds(s * RPT, RPT)], degv)

    def scal(i, _):
        sl = pl.ds(i * 16, 16)
        dg = degv[sl] + 1.0
        d2 = (1.0 - ALPHA) / dg
        xi = plsc.bitcast(dg, jnp.int32)
        yi = 0x5F3759DF - (xi >> 1)
        yv = plsc.bitcast(yi, jnp.float32)
        for _n in range(3):
            yv = yv * (1.5 - 0.5 * dg * yv * yv)
        dsq = dg * yv
        d2v[sl] = d2
        f1v[sl] = dsq * d2
        f2v[sl] = ALPHA * dsq
        degv[sl] = yv
        return _

    lax.fori_loop(0, RPT // 16, scal, None)

    for b in range(NB_INIT):
        r0 = s * RPT + b * CH
        pltpu.sync_copy(y0c.at[c, pl.ds(r0, CH)], rows.at[0])

        def g0row(i, _, b=b):
            dsc = _splat(degv, b * CH + i)
            for half in range(HC // 16):
                sl = pl.ds(half * 16, 16)
                rows.at[1][i, sl] = dsc * rows.at[0][i, sl]
            return _

        lax.fori_loop(0, CH, g0row, None)
        pltpu.sync_copy(rows.at[1], g0b.at[c, pl.ds(r0, CH)])
        pltpu.sync_copy(rows.at[1], acc.at[pl.ds(r0, CH)])
    plsc.subcore_barrier()

    for r in range(K):
        gsrc = g0b if r == 0 else gbuf
        gdst = gbuf if r < K - 1 else y3c

        for b in range(NBUF):
            pltpu.async_copy(gsrc.at[c].at[sidx.at[b]], rows.at[b], gsem[b])

        def body(t, _, gsrc=gsrc):
            for b in range(NBUF):
                j = t * NBUF + b
                pltpu.make_async_copy(gsrc.at[c].at[sidx.at[0]], rows.at[b],
                                      gsem[b]).wait()
                pltpu.async_copy(rows.at[b], acc.at[didx.at[j]], ssem[b],
                                 add=True)
            for b in range(NBUF):
                pltpu.make_async_copy(rows.at[b], acc.at[didx.at[0]],
                                      ssem[b]).wait()

                @pl.when(t < NGRP - 1)
                def _():
                    pltpu.async_copy(gsrc.at[c].at[sidx.at[t * NBUF + b + NBUF]],
                                     rows.at[b], gsem[b])

            return _

        lax.fori_loop(0, NGRP, body, None)
        plsc.subcore_barrier()

        for b in range(NB_INIT):
            r0 = s * RPT + b * CH
            pltpu.sync_copy(acc.at[pl.ds(r0, CH)], rows.at[0])
            pltpu.sync_copy(g0b.at[c, pl.ds(r0, CH)], rows.at[1])

            def crow(i, _, b=b, r=r):
                if r < K - 1:
                    a_s = _splat(d2v, b * CH + i)
                    b_s = ALPHA
                else:
                    a_s = _splat(f1v, b * CH + i)
                    b_s = _splat(f2v, b * CH + i)
                for half in range(HC // 16):
                    sl = pl.ds(half * 16, 16)
                    rows.at[2][i, sl] = (a_s * rows.at[0][i, sl]
                                         + b_s * rows.at[1][i, sl])
                return _

            lax.fori_loop(0, CH, crow, None)
            pltpu.sync_copy(rows.at[2], gdst.at[c, pl.ds(r0, CH)])
            if r < K - 1:
                pltpu.sync_copy(rows.at[2], acc.at[pl.ds(r0, CH)])
        plsc.subcore_barrier()


@functools.cache
def _appnp_sc():
    return pl.kernel(
        _appnp_body,
        out_type=(jax.ShapeDtypeStruct((NC, NR, HC), jnp.float32),
                  jax.ShapeDtypeStruct((NC, NR, HC), jnp.float32),
                  jax.ShapeDtypeStruct((NC, NR, HC), jnp.float32)),
        mesh=_mesh(),
        scratch_types=[
            pltpu.VMEM((NCHT, CH), jnp.int32),
            pltpu.VMEM((NCHT, CH), jnp.int32),
            pltpu.VMEM((NBUF, CH, HC), jnp.float32),
            pltpu.VMEM((CH,), jnp.float32),
            pltpu.VMEM((RPT,), jnp.float32),
            pltpu.VMEM((RPT,), jnp.float32),
            pltpu.VMEM((RPT,), jnp.float32),
            pltpu.VMEM((RPT,), jnp.float32),
            pltpu.VMEM_SHARED((NR, HC), jnp.float32),
            pltpu.VMEM_SHARED((NR,), jnp.float32),
            pltpu.SemaphoreType.DMA,
            [pltpu.SemaphoreType.DMA] * NBUF,
            [pltpu.SemaphoreType.DMA] * NBUF,
        ],
        compiler_params=pltpu.CompilerParams(use_tc_tiling_on_sc=False,
                                             needs_layout_passes=False),
    )


_BR = 512
_NBLK = NR // _BR


def _prep_body(x_ref, w1_ref, y_ref):
    y = jnp.dot(x_ref[...], w1_ref[...], preferred_element_type=jnp.float32)
    y_ref[0] = y[:, :HC]
    y_ref[1] = y[:, HC:]


def _prep_tc(xp, w1):
    return pl.pallas_call(
        _prep_body,
        grid=(_NBLK,),
        in_specs=[
            pl.BlockSpec((_BR, D), lambda i: (i, 0)),
            pl.BlockSpec((D, H), lambda i: (0, 0)),
        ],
        out_specs=pl.BlockSpec((NC, _BR, HC), lambda i: (0, i, 0)),
        out_shape=jax.ShapeDtypeStruct((NC, NR, HC), jnp.float32),
    )(xp, w1)


def _final_body(y3_ref, batch_ref, b1_ref, w2_ref, b2_ref, w3_ref,
                b3_ref, out_ref, accs, accc):
    pid = pl.program_id(0)

    @pl.when(pid == 0)
    def _():
        accs[...] = jnp.zeros_like(accs)
        accc[...] = jnp.zeros_like(accc)

    y3 = jnp.concatenate([y3_ref[0], y3_ref[1]], axis=1)
    z = jnp.maximum(y3 + b1_ref[...], 0.0)
    onehot = (batch_ref[...] ==
              lax.broadcasted_iota(jnp.int32, (_BR, G), 1)).astype(jnp.float32)
    accs[...] += lax.dot_general(onehot, z, (((0,), (0,)), ((), ())),
                                 preferred_element_type=jnp.float32)
    accc[...] += lax.dot_general(onehot, jnp.ones((_BR, 1), jnp.float32),
                                 (((0,), (0,)), ((), ())),
                                 preferred_element_type=jnp.float32)

    @pl.when(pid == _NBLK - 1)
    def _():
        pooled = accs[...] / jnp.maximum(accc[...], 1.0)
        a = jnp.maximum(
            jnp.dot(pooled, w2_ref[...], preferred_element_type=jnp.float32)
            + b2_ref[...], 0.0)
        out_ref[...] = (
            jnp.dot(a, w3_ref[...], preferred_element_type=jnp.float32)
            + b3_ref[...])


def _final_tc(y3c, batchp, b1, w2, b2, w3, b3):
    return pl.pallas_call(
        _final_body,
        grid=(_NBLK,),
        in_specs=[
            pl.BlockSpec((NC, _BR, HC), lambda i: (0, i, 0)),
            pl.BlockSpec((_BR, 1), lambda i: (i, 0)),
            pl.BlockSpec((1, H), lambda i: (0, 0)),
            pl.BlockSpec((H, ATT), lambda i: (0, 0)),
            pl.BlockSpec((1, ATT), lambda i: (0, 0)),
            pl.BlockSpec((ATT, 1), lambda i: (0, 0)),
            pl.BlockSpec((1, 1), lambda i: (0, 0)),
        ],
        out_specs=pl.BlockSpec((G, 1), lambda i: (0, 0)),
        out_shape=jax.ShapeDtypeStruct((G, 1), jnp.float32),
        scratch_shapes=[
            pltpu.VMEM((G, H), jnp.float32),
            pltpu.VMEM((G, 1), jnp.float32),
        ],
    )(y3c, batchp, b1, w2, b2, w3, b3)


def kernel(x, edge_index, batch, W1, b1, W2, b2, W3, b3):
    pad_ids = DUMP0 + (jnp.arange(PAD_E - E, dtype=jnp.int32) % CH)
    srcp = jnp.concatenate([edge_index[0], pad_ids]).reshape(NS * NCHT, CH)
    dstp = jnp.concatenate([edge_index[1], pad_ids]).reshape(NS * NCHT, CH)
    xp = jnp.pad(x, ((0, NR - N), (0, 0)))
    batchp = jnp.pad(batch, (0, NR - N), constant_values=G).reshape(NR, 1)

    y0c = _prep_tc(xp, W1)
    y3c, _, _ = _appnp_sc()(y0c, srcp, dstp)
    out = _final_tc(y3c, batchp, b1.reshape(1, H), W2,
                    b2.reshape(1, ATT), W3, b3.reshape(1, 1))
    return out
